# Initial kernel scaffold; baseline (speedup 1.0000x reference)
import jax
import jax.numpy as jnp
from jax.experimental import pallas as pl


def kernel(x, edge_index, batch, W0, a_src0, a_dst0, b0, W1, a_src1, a_dst1, b1, W2, a_src2, a_dst2, b2, Wc, bc):
    raise NotImplementedError("write your pallas kernel here")


# speedup vs baseline: 11.3037x; 11.3037x over previous
import functools

import jax
import jax.numpy as jnp
from jax import lax
from jax.experimental import pallas as pl
from jax.experimental.pallas import tpu as pltpu
from jax.experimental.pallas import tpu_sc as plsc

N = 10000
E = 320000
EL = E + N
NF = 128
HID = 128
HEADS = 8
NG = 64
NC = 4

B = 128
NP = 10112
EPAD = 331776
ZR = NP // 16
WR = 624

_f32 = jnp.float32
_i32 = jnp.int32


def _iota16():
    return lax.broadcasted_iota(_i32, (16,), 0)


def _full16(v):
    return jnp.full((16,), v, dtype=_i32)


def _edge_range(c, s, split32):
    if split32:
        span = EPAD // 32
        tbase = (s * 2 + c) * span
    else:
        span = EPAD // 16
        tbase = s * span
    return tbase, span // B


def _wr_out(src_ref, dst_ref, s):
    @pl.when(s < 15)
    def _():
        pltpu.sync_copy(src_ref.at[pl.ds(s * WR, WR)],
                        dst_ref.at[pl.ds(s * WR, WR)])

    @pl.when(s == 15)
    def _():
        pltpu.sync_copy(src_ref.at[pl.ds(15 * WR, N - 15 * WR)],
                        dst_ref.at[pl.ds(15 * WR, N - 15 * WR)])


def _den_sc_body(src_hbm, dst_hbm, sd_hbm, den_hbm,
                 idxs, idxd, sidx, didx, sbuf, dbuf, dnv, den,
                 sem0, sem1, *, heads_of, split32):
    c = lax.axis_index("c")
    s = lax.axis_index("s")
    tbase, nblk = _edge_range(c, s, split32)
    lane = _iota16()

    def zrow(i, _):
        dnv[i, pl.ds(0, 16)] = jnp.zeros((16,), _f32)
        return 0

    lax.fori_loop(0, B, zrow, 0)
    zs = s * ZR
    for z in range(ZR // B):
        pltpu.sync_copy(dnv, den.at[pl.ds(zs + z * B, B)])
    plsc.subcore_barrier()

    def blk_body(blk, _):
        base = tbase + blk * B
        pltpu.sync_copy(src_hbm.at[pl.ds(base, B)], idxs)
        pltpu.sync_copy(dst_hbm.at[pl.ds(base, B)], idxd)
        lax.fori_loop(0, B, zrow, 0)
        for j in heads_of:
            hd = c * len(heads_of) + j if not split32 else 0

            def iix(k, _):
                sl = pl.ds(k * 16, 16)
                sidx[sl] = idxs[sl] * 16 + hd
                didx[sl] = idxd[sl] * 16 + (8 + hd)
                return 0

            lax.fori_loop(0, B // 16, iix, 0)
            d0 = pltpu.async_copy(sd_hbm.at[sidx], sbuf, sem0)
            d1 = pltpu.async_copy(sd_hbm.at[didx], dbuf, sem1)
            d0.wait()
            d1.wait()

            def grp(g, _):
                sl = pl.ds(g * 16, 16)
                v = sbuf[sl] + dbuf[sl]
                v = jnp.where(v > 0, v, 0.2 * v)
                exv = jnp.exp(v)
                ml = lane == hd
                for i2 in range(16):
                    i = g * 16 + i2
                    exi = jnp.full((16,), exv[i2], _f32)
                    dnv[i, pl.ds(0, 16)] = (
                        dnv[i, pl.ds(0, 16)] + jnp.where(ml, exi, 0.0))
                return 0

            lax.fori_loop(0, B // 16, grp, 0)
        pltpu.sync_copy(dnv, den.at[idxd], add=True)
        return 0

    lax.fori_loop(0, nblk, blk_body, 0)
    plsc.subcore_barrier()
    _wr_out(den, den_hbm.at[c], s)


def _acc_sc_body(src_hbm, dst_hbm, sd_hbm, h_hbm, out_hbm,
                 idxs, idxd, idxh, sidx, didx, sbuf, dbuf, hv, ov, acc,
                 sem0, sem1, sem2, *, hpc, split32, hmul):
    c = lax.axis_index("c")
    s = lax.axis_index("s")
    tbase, nblk = _edge_range(c, s, split32)

    def zrow(i, _):
        z = jnp.zeros((16,), _f32)
        for k in range(8):
            ov[i, pl.ds(k * 16, 16)] = z
        return 0

    lax.fori_loop(0, B, zrow, 0)

    for j in range(hpc):
        if split32:
            hd = 0
            oslab = c
        else:
            hd = c * hpc + j
            oslab = hd

        zs = s * ZR
        for z in range(ZR // B):
            pltpu.sync_copy(ov, acc.at[pl.ds(zs + z * B, B)])
        plsc.subcore_barrier()

        def blk_body(blk, _):
            base = tbase + blk * B
            pltpu.sync_copy(src_hbm.at[pl.ds(base, B)], idxs)
            pltpu.sync_copy(dst_hbm.at[pl.ds(base, B)], idxd)

            def iix(k, _):
                sl = pl.ds(k * 16, 16)
                sidx[sl] = idxs[sl] * 16 + hd
                didx[sl] = idxd[sl] * 16 + (8 + hd)
                if hmul != 1:
                    idxh[sl] = idxs[sl] * hmul + hd
                return 0

            lax.fori_loop(0, B // 16, iix, 0)
            hidx = idxs if hmul == 1 else idxh
            d0 = pltpu.async_copy(sd_hbm.at[sidx], sbuf, sem0)
            d1 = pltpu.async_copy(sd_hbm.at[didx], dbuf, sem1)
            d2 = pltpu.async_copy(h_hbm.at[hidx], hv, sem2)
            d0.wait()
            d1.wait()
            d2.wait()

            def grp(g, _):
                sl = pl.ds(g * 16, 16)
                v = sbuf[sl] + dbuf[sl]
                v = jnp.where(v > 0, v, 0.2 * v)
                exv = jnp.exp(v)
                for i2 in range(16):
                    ex = jnp.full((16,), exv[i2], _f32)
                    i = g * 16 + i2
                    for k in range(8):
                        ov[i, pl.ds(k * 16, 16)] = (
                            hv[i, pl.ds(k * 16, 16)] * ex)
                return 0

            lax.fori_loop(0, B // 16, grp, 0)
            pltpu.sync_copy(ov, acc.at[idxd], add=True)
            return 0

        lax.fori_loop(0, nblk, blk_body, 0)
        plsc.subcore_barrier()
        _wr_out(acc, out_hbm.at[oslab], s)
        plsc.subcore_barrier()


@functools.cache
def _make_sc_den(nheads, split32):
    mesh = plsc.VectorSubcoreMesh(core_axis_name="c", subcore_axis_name="s",
                                  num_cores=2, num_subcores=16)
    body = functools.partial(_den_sc_body, heads_of=tuple(range(nheads)),
                             split32=split32)
    return pl.kernel(
        body,
        out_type=jax.ShapeDtypeStruct((2, N, 16), _f32),
        mesh=mesh,
        scratch_types=[
            pltpu.VMEM((B,), _i32),
            pltpu.VMEM((B,), _i32),
            pltpu.VMEM((B,), _i32),
            pltpu.VMEM((B,), _i32),
            pltpu.VMEM((B,), _f32),
            pltpu.VMEM((B,), _f32),
            pltpu.VMEM((B, 16), _f32),
            pltpu.VMEM_SHARED((NP, 16), _f32),
            pltpu.SemaphoreType.DMA,
            pltpu.SemaphoreType.DMA,
        ],
    )


@functools.cache
def _make_sc_acc(n_out_slabs, hpc, split32, hmul):
    mesh = plsc.VectorSubcoreMesh(core_axis_name="c", subcore_axis_name="s",
                                  num_cores=2, num_subcores=16)
    body = functools.partial(_acc_sc_body, hpc=hpc, split32=split32,
                             hmul=hmul)
    return pl.kernel(
        body,
        out_type=jax.ShapeDtypeStruct((n_out_slabs, N, 128), _f32),
        mesh=mesh,
        scratch_types=[
            pltpu.VMEM((B,), _i32),
            pltpu.VMEM((B,), _i32),
            pltpu.VMEM((B,), _i32),
            pltpu.VMEM((B,), _i32),
            pltpu.VMEM((B,), _i32),
            pltpu.VMEM((B,), _f32),
            pltpu.VMEM((B,), _f32),
            pltpu.VMEM((B, 128), _f32),
            pltpu.VMEM((B, 128), _f32),
            pltpu.VMEM_SHARED((NP, 128), _f32),
            pltpu.SemaphoreType.DMA,
            pltpu.SemaphoreType.DMA,
            pltpu.SemaphoreType.DMA,
        ],
    )


def _sc_layer8(srcp, dstp, sdp, hview):
    den = _make_sc_den(4, False)(srcp, dstp, sdp)
    acc = _make_sc_acc(8, 4, False, 8)(srcp, dstp, sdp, hview)
    return acc, den


def _sc_layer1(srcp, dstp, sdp, hview):
    den = _make_sc_den(1, True)(srcp, dstp, sdp)
    acc = _make_sc_acc(2, 1, True, 1)(srcp, dstp, sdp, hview)
    return acc, den


def _mm0_body(x_ref, w_ref, asd_ref, h_ref, sd_ref):
    h = jnp.dot(x_ref[...], w_ref[...], preferred_element_type=_f32)
    h_ref[...] = h
    sd_ref[...] = jnp.dot(h, asd_ref[...], preferred_element_type=_f32)


def _mm0(x, w, asd):
    return pl.pallas_call(
        _mm0_body,
        grid=(10,),
        in_specs=[
            pl.BlockSpec((1000, NF), lambda n: (n, 0)),
            pl.BlockSpec((NF, 1024), lambda n: (0, 0)),
            pl.BlockSpec((1024, 16), lambda n: (0, 0)),
        ],
        out_specs=[
            pl.BlockSpec((1000, 1024), lambda n: (n, 0)),
            pl.BlockSpec((1000, 16), lambda n: (n, 0)),
        ],
        out_shape=[
            jax.ShapeDtypeStruct((N, 1024), _f32),
            jax.ShapeDtypeStruct((N, 16), _f32),
        ],
    )(x, w, asd)


def _mmk_body(x_ref, w_ref, asd_ref, h_ref, sd_ref):
    k = pl.program_id(1)

    @pl.when(k == 0)
    def _():
        h_ref[...] = jnp.zeros_like(h_ref)

    h_ref[...] += jnp.dot(x_ref[...], w_ref[...], preferred_element_type=_f32)

    @pl.when(k == pl.num_programs(1) - 1)
    def _():
        sd_ref[...] = jnp.dot(h_ref[...], asd_ref[...],
                              preferred_element_type=_f32)


def _mmk(x, w, asd, out_w):
    return pl.pallas_call(
        _mmk_body,
        grid=(10, 8),
        in_specs=[
            pl.BlockSpec((1000, 128), lambda n, k: (n, k)),
            pl.BlockSpec((128, out_w), lambda n, k: (k, 0)),
            pl.BlockSpec((out_w, 16), lambda n, k: (0, 0)),
        ],
        out_specs=[
            pl.BlockSpec((1000, out_w), lambda n, k: (n, 0)),
            pl.BlockSpec((1000, 16), lambda n, k: (n, 0)),
        ],
        out_shape=[
            jax.ShapeDtypeStruct((N, out_w), _f32),
            jax.ShapeDtypeStruct((N, 16), _f32),
        ],
    )(x, w, asd)


def _norm_body(acc_ref, d0_ref, d1_ref, b_ref, o_ref):
    h = pl.program_id(0)
    oh = (lax.broadcasted_iota(_i32, (16, 1), 0) == h).astype(_f32)
    dn = jnp.dot(d0_ref[...] + d1_ref[...], oh,
                 preferred_element_type=_f32) + 1e-16
    v = acc_ref[0] / dn + b_ref[0]
    o_ref[...] = jnp.where(v > 0, v, jnp.exp(jnp.minimum(v, 0.0)) - 1.0)


def _norm(acc, d0, d1, b2d):
    b2d = b2d.reshape(8, 1, 128)
    return pl.pallas_call(
        _norm_body,
        grid=(8, 10),
        in_specs=[
            pl.BlockSpec((1, 1000, 128), lambda h, n: (h, n, 0)),
            pl.BlockSpec((1000, 16), lambda h, n: (n, 0)),
            pl.BlockSpec((1000, 16), lambda h, n: (n, 0)),
            pl.BlockSpec((1, 1, 128), lambda h, n: (h, 0, 0)),
        ],
        out_specs=pl.BlockSpec((1000, 128), lambda h, n: (n, h)),
        out_shape=jax.ShapeDtypeStruct((N, 1024), _f32),
    )(acc, d0, d1, b2d)


def _final_body(a0_ref, a1_ref, d0_ref, d1_ref, b_ref, bt_ref, wc_ref,
                bc_ref, o_ref, psum, cnt):
    n = pl.program_id(0)

    @pl.when(n == 0)
    def _():
        psum[...] = jnp.zeros_like(psum)
        cnt[...] = jnp.zeros_like(cnt)

    e0 = (lax.broadcasted_iota(_i32, (16, 1), 0) == 0).astype(_f32)
    dn = jnp.dot(d0_ref[...] + d1_ref[...], e0,
                 preferred_element_type=_f32) + 1e-16
    v = (a0_ref[...] + a1_ref[...]) / dn + b_ref[...]
    xf = jnp.where(v > 0, v, jnp.exp(jnp.minimum(v, 0.0)) - 1.0)
    oh = (bt_ref[...] == lax.broadcasted_iota(_i32, (1, NG), 1)).astype(_f32)
    dnum = (((0,), (0,)), ((), ()))
    psum[...] += lax.dot_general(oh, xf, dnum, preferred_element_type=_f32)
    cnt[...] += lax.dot_general(oh, jnp.ones_like(xf), dnum,
                                preferred_element_type=_f32)

    @pl.when(n == pl.num_programs(0) - 1)
    def _():
        pooled = psum[...] / jnp.maximum(cnt[...], 1.0)
        o_ref[...] = jnp.dot(pooled, wc_ref[...],
                             preferred_element_type=_f32) + bc_ref[...]


def _final(a0, a1, d0, d1, b2d, bt2d, wcp, bcp):
    return pl.pallas_call(
        _final_body,
        grid=(10,),
        in_specs=[
            pl.BlockSpec((1000, 128), lambda n: (n, 0)),
            pl.BlockSpec((1000, 128), lambda n: (n, 0)),
            pl.BlockSpec((1000, 16), lambda n: (n, 0)),
            pl.BlockSpec((1000, 16), lambda n: (n, 0)),
            pl.BlockSpec((1, 128), lambda n: (0, 0)),
            pl.BlockSpec((1000, 1), lambda n: (n, 0)),
            pl.BlockSpec((128, 128), lambda n: (0, 0)),
            pl.BlockSpec((1, 128), lambda n: (0, 0)),
        ],
        out_specs=pl.BlockSpec((NG, 128), lambda n: (0, 0)),
        out_shape=jax.ShapeDtypeStruct((NG, 128), _f32),
        scratch_shapes=[
            pltpu.VMEM((NG, 128), _f32),
            pltpu.VMEM((NG, 128), _f32),
        ],
    )(a0, a1, d0, d1, b2d, bt2d, wcp, bcp)


def _mk_asd(a_src, a_dst):
    heads = a_src.shape[0]
    if heads == 1:
        z = jnp.zeros((HID, 7), _f32)
        return jnp.concatenate(
            [a_src.T, z, a_dst.T, z], axis=1)
    eye = jnp.eye(heads, dtype=_f32)
    a_s = (a_src[:, :, None] * eye[:, None, :]).reshape(heads * HID, heads)
    a_d = (a_dst[:, :, None] * eye[:, None, :]).reshape(heads * HID, heads)
    return jnp.concatenate([a_s, a_d], axis=1)


def _pad_sd(sd):
    return jnp.concatenate(
        [sd, jnp.zeros((NP - N, 16), _f32)], axis=0).reshape(NP * 16)


def kernel(x, edge_index, batch, W0, a_src0, a_dst0, b0, W1, a_src1, a_dst1,
           b1, W2, a_src2, a_dst2, b2, Wc, bc):
    loops = jnp.arange(N, dtype=_i32)
    npad = EPAD - EL
    srcp = jnp.concatenate(
        [edge_index[0].astype(_i32), loops, jnp.zeros((npad,), _i32)])
    dstp = jnp.concatenate(
        [edge_index[1].astype(_i32), loops, jnp.full((npad,), N, _i32)])
    bt2d = batch.astype(_i32).reshape(N, 1)

    asd0 = _mk_asd(a_src0, a_dst0)
    asd1 = _mk_asd(a_src1, a_dst1)
    asd2 = _mk_asd(a_src2, a_dst2)
    wcp = jnp.concatenate([Wc, jnp.zeros((HID, 128 - NC), _f32)], axis=1)
    bcp = jnp.concatenate([bc, jnp.zeros((128 - NC,), _f32)]).reshape(1, 128)

    h0, sd0 = _mm0(x, W0, asd0)
    acc0, den0 = _sc_layer8(srcp, dstp, _pad_sd(sd0), h0.reshape(N * 8, 128))
    x1 = _norm(acc0, den0[0], den0[1], b0.reshape(8, 128))

    h1, sd1 = _mmk(x1, W1, asd1, 1024)
    acc1, den1 = _sc_layer8(srcp, dstp, _pad_sd(sd1), h1.reshape(N * 8, 128))
    x2 = _norm(acc1, den1[0], den1[1], b1.reshape(8, 128))

    h2, sd2 = _mmk(x2, W2, asd2, 128)
    acc2, den2 = _sc_layer1(srcp, dstp, _pad_sd(sd2), h2)

    out = _final(acc2[0], acc2[1], den2[0], den2[1], b2.reshape(1, 128),
                 bt2d, wcp, bcp)
    return out[:, :NC]
